# baseline (speedup 1.0000x reference)
import functools

import jax
import jax.numpy as jnp
import numpy as np
from jax import lax
from jax.experimental import pallas as pl
from jax.experimental.pallas import tpu as pltpu
from jax.experimental.pallas import tpu_sc as plsc

_VOCAB_SIZES = [100003, 100019, 100043, 100049, 100057, 100069, 100103, 100109]
_OFFSETS = np.cumsum([0] + _VOCAB_SIZES[:-1]).astype(np.int32)

_NUM_CORES = 2
_NUM_SUBCORES = 16
_NUM_WORKERS = _NUM_CORES * _NUM_SUBCORES
_LANES = 16
_CHUNK = 128
_GROUPS = 4


def _offset_table():
    rows = []
    for j in range(_GROUPS):
        rows.append(np.tile([_OFFSETS[j], _OFFSETS[j + _GROUPS]], _LANES // 2))
    return np.asarray(rows, dtype=np.int32)


@functools.partial(jax.jit, static_argnames=("n", "d"))
def _mhe_lookup(ids_r, off_tbl, table, *, n, d):
    n_per_w = n // _NUM_WORKERS
    rows_per_w = n_per_w // _GROUPS
    chunks_per_group = rows_per_w // _CHUNK
    idx_rows = n_per_w // _CHUNK
    mesh = plsc.VectorSubcoreMesh(core_axis_name="c", subcore_axis_name="s")

    @functools.partial(
        pl.kernel,
        mesh=mesh,
        out_type=jax.ShapeDtypeStruct((n // _GROUPS, _GROUPS * d), jnp.float32),
        scratch_types=[
            pltpu.VMEM((idx_rows, _CHUNK), jnp.int32),
            pltpu.VMEM((_GROUPS, _LANES), jnp.int32),
            pltpu.VMEM((rows_per_w, d), jnp.float32),
            pltpu.SemaphoreType.DMA,
        ],
        compiler_params=pltpu.CompilerParams(use_tc_tiling_on_sc=False),
    )
    def k(ids_hbm, off_hbm, table_hbm, out_hbm, idx_v, off_v, rows_v, sem):
        wid = lax.axis_index("s") * _NUM_CORES + lax.axis_index("c")
        pltpu.sync_copy(ids_hbm.at[pl.ds(wid * idx_rows, idx_rows)], idx_v)
        pltpu.sync_copy(off_hbm, off_v)

        for j in range(_GROUPS):
            off = off_v[j]
            for c in range(chunks_per_group):
                row = j * chunks_per_group + c
                for t in range(_CHUNK // _LANES):
                    sl = pl.ds(t * _LANES, _LANES)
                    idx_v[row, sl] = idx_v[row, sl] + off

        out_base = wid * rows_per_w
        for j in range(_GROUPS):
            copies = []
            for c in range(chunks_per_group):
                row = j * chunks_per_group + c
                copies.append(
                    pltpu.async_copy(
                        table_hbm.at[idx_v.at[row]],
                        rows_v.at[pl.ds(c * _CHUNK, _CHUNK)],
                        sem,
                    )
                )
            for cp in copies:
                cp.wait()
            pltpu.sync_copy(
                rows_v,
                out_hbm.at[pl.ds(out_base, rows_per_w), pl.ds(j * d, d)],
            )

    return k(ids_r, off_tbl, table)


def kernel(input_ids, table):
    b, s, h = input_ids.shape
    d = table.shape[1]
    n = b * s * h
    ids_r = (
        input_ids.reshape(_NUM_WORKERS, n // (_NUM_WORKERS * _GROUPS), _GROUPS)
        .transpose(0, 2, 1)
        .reshape(n // 128, 128)
    )
    off_tbl = jnp.asarray(_offset_table())
    out = _mhe_lookup(ids_r, off_tbl, table, n=n, d=d)
    return out.reshape(b, s, h, d)
